# Initial kernel scaffold; baseline (speedup 1.0000x reference)
import jax
import jax.numpy as jnp
from jax.experimental import pallas as pl


def kernel(hidden_states, gate_weight, correction_bias, w_gate, w_up, w_down, sw_gate, sw_up, sw_down, num_global_tokens, max_num_tokens_per_gpu):
    raise NotImplementedError("write your pallas kernel here")


# speedup vs baseline: 1.7850x; 1.7850x over previous
import jax
import jax.numpy as jnp
from jax.experimental import pallas as pl
from jax.experimental.pallas import tpu as pltpu

E = 16
TOPK = 2
NGROUP = 4
GSIZE = E // NGROUP
H = 2048
I = 1024
RSF = 2.5
NEG = -1e30
IB = 512


def _silu(x):
    return x * jax.nn.sigmoid(x)


def _bf16_dot(a, b, dn):
    return jax.lax.dot_general(
        a.astype(jnp.bfloat16), b.astype(jnp.bfloat16), dn,
        preferred_element_type=jnp.float32)


_DN_T = (((1,), (1,)), ((), ()))


def _routing_combine(x, gate_w, bias):
    logits = jax.lax.dot_general(x, gate_w, _DN_T,
                                 preferred_element_type=jnp.float32)
    scores = jax.nn.sigmoid(logits)
    sfc = scores + bias
    T = x.shape[0]
    iota = jax.lax.broadcasted_iota(jnp.int32, (T, E), 1)
    gid = iota // GSIZE

    gsum = jnp.zeros((T, E), jnp.float32)
    for g in range(NGROUP):
        sg = jnp.where(gid == g, sfc, NEG)
        m1 = jnp.max(sg, axis=1, keepdims=True)
        i1 = jnp.min(jnp.where(sg == m1, iota, E), axis=1, keepdims=True)
        m2 = jnp.max(jnp.where(iota == i1, NEG, sg), axis=1, keepdims=True)
        gsum = jnp.where(gid == g, m1 + m2, gsum)

    g1v = jnp.max(gsum, axis=1, keepdims=True)
    g1 = jnp.min(jnp.where(gsum == g1v, gid, NGROUP), axis=1, keepdims=True)
    rem = jnp.where(gid == g1, NEG, gsum)
    g2v = jnp.max(rem, axis=1, keepdims=True)
    g2 = jnp.min(jnp.where(rem == g2v, gid, NGROUP), axis=1, keepdims=True)
    selected = (gid == g1) | (gid == g2)

    ms = jnp.where(selected, sfc, 0.0)
    m1 = jnp.max(ms, axis=1, keepdims=True)
    e1 = jnp.min(jnp.where(ms == m1, iota, E), axis=1, keepdims=True)
    ms2 = jnp.where(iota == e1, NEG, ms)
    m2 = jnp.max(ms2, axis=1, keepdims=True)
    e2 = jnp.min(jnp.where(ms2 == m2, iota, E), axis=1, keepdims=True)

    w1 = jnp.sum(jnp.where(iota == e1, scores, 0.0), axis=1, keepdims=True)
    w2 = jnp.sum(jnp.where(iota == e2, scores, 0.0), axis=1, keepdims=True)
    denom = w1 + w2 + 1e-20
    w1n = w1 / denom * RSF
    w2n = w2 / denom * RSF
    return (jnp.where(iota == e1, w1n, 0.0)
            + jnp.where(iota == e2, w2n, 0.0))


def _moe_kernel(x_ref, gw_ref, bias_ref, wg_ref, wu_ref, wd_ref,
                swg_ref, swu_ref, swd_ref, out_ref, we_ref):
    e = pl.program_id(0)
    ki = pl.program_id(1)
    x = x_ref[...]

    @pl.when((e == 0) & (ki == 0))
    def _first():
        we_ref[...] = _routing_combine(x, gw_ref[...], bias_ref[...])
        sg = _bf16_dot(x, swg_ref[...], _DN_T)
        su = _bf16_dot(x, swu_ref[...], _DN_T)
        out_ref[...] = _bf16_dot(_silu(sg) * su, swd_ref[...], _DN_T)

    g = _bf16_dot(x, wg_ref[0], _DN_T)
    u = _bf16_dot(x, wu_ref[0], _DN_T)
    h = _silu(g) * u
    y = jax.lax.dot_general(h.astype(jnp.bfloat16),
                            wd_ref[0].astype(jnp.bfloat16),
                            (((1,), (1,)), ((), ())),
                            preferred_element_type=jnp.float32)
    iota = jax.lax.broadcasted_iota(jnp.int32, we_ref.shape, 1)
    w_col = jnp.sum(jnp.where(iota == e, we_ref[...], 0.0), axis=1,
                    keepdims=True)
    out_ref[...] += w_col * y


def kernel(hidden_states, gate_weight, correction_bias, w_gate, w_up, w_down,
           sw_gate, sw_up, sw_down, num_global_tokens, max_num_tokens_per_gpu):
    T = hidden_states.shape[0]
    bias2d = correction_bias.reshape(1, E)
    n_ki = I // IB
    grid = (E, n_ki)
    return pl.pallas_call(
        _moe_kernel,
        grid=grid,
        in_specs=[
            pl.BlockSpec((T, H), lambda e, k: (0, 0)),
            pl.BlockSpec((E, H), lambda e, k: (0, 0)),
            pl.BlockSpec((1, E), lambda e, k: (0, 0)),
            pl.BlockSpec((1, IB, H), lambda e, k: (e, k, 0)),
            pl.BlockSpec((1, IB, H), lambda e, k: (e, k, 0)),
            pl.BlockSpec((1, H, IB), lambda e, k: (e, 0, k)),
            pl.BlockSpec((I, H), lambda e, k: (0, 0)),
            pl.BlockSpec((I, H), lambda e, k: (0, 0)),
            pl.BlockSpec((H, I), lambda e, k: (0, 0)),
        ],
        out_specs=pl.BlockSpec((T, H), lambda e, k: (0, 0)),
        out_shape=jax.ShapeDtypeStruct((T, H), jnp.float32),
        scratch_shapes=[pltpu.VMEM((T, E), jnp.float32)],
        compiler_params=pltpu.CompilerParams(
            dimension_semantics=("arbitrary", "arbitrary")),
    )(hidden_states, gate_weight, bias2d, w_gate, w_up, w_down,
      sw_gate, sw_up, sw_down)


# speedup vs baseline: 1.7921x; 1.0040x over previous
import jax
import jax.numpy as jnp
from jax.experimental import pallas as pl
from jax.experimental.pallas import tpu as pltpu

E = 16
TOPK = 2
NGROUP = 4
GSIZE = E // NGROUP
H = 2048
I = 1024
RSF = 2.5
NEG = -1e30
IB = 512


def _silu(x):
    return x * jax.nn.sigmoid(x)


def _bf16_dot(a, b, dn):
    return jax.lax.dot_general(a, b, dn,
                               precision=jax.lax.Precision.DEFAULT,
                               preferred_element_type=jnp.float32)


_DN_T = (((1,), (1,)), ((), ()))


def _routing_combine(x, gate_w, bias):
    logits = jax.lax.dot_general(x, gate_w, _DN_T,
                                 preferred_element_type=jnp.float32)
    scores = jax.nn.sigmoid(logits)
    sfc = scores + bias
    T = x.shape[0]
    iota = jax.lax.broadcasted_iota(jnp.int32, (T, E), 1)
    gid = iota // GSIZE

    gsum = jnp.zeros((T, E), jnp.float32)
    for g in range(NGROUP):
        sg = jnp.where(gid == g, sfc, NEG)
        m1 = jnp.max(sg, axis=1, keepdims=True)
        i1 = jnp.min(jnp.where(sg == m1, iota, E), axis=1, keepdims=True)
        m2 = jnp.max(jnp.where(iota == i1, NEG, sg), axis=1, keepdims=True)
        gsum = jnp.where(gid == g, m1 + m2, gsum)

    g1v = jnp.max(gsum, axis=1, keepdims=True)
    g1 = jnp.min(jnp.where(gsum == g1v, gid, NGROUP), axis=1, keepdims=True)
    rem = jnp.where(gid == g1, NEG, gsum)
    g2v = jnp.max(rem, axis=1, keepdims=True)
    g2 = jnp.min(jnp.where(rem == g2v, gid, NGROUP), axis=1, keepdims=True)
    selected = (gid == g1) | (gid == g2)

    ms = jnp.where(selected, sfc, 0.0)
    m1 = jnp.max(ms, axis=1, keepdims=True)
    e1 = jnp.min(jnp.where(ms == m1, iota, E), axis=1, keepdims=True)
    ms2 = jnp.where(iota == e1, NEG, ms)
    m2 = jnp.max(ms2, axis=1, keepdims=True)
    e2 = jnp.min(jnp.where(ms2 == m2, iota, E), axis=1, keepdims=True)

    w1 = jnp.sum(jnp.where(iota == e1, scores, 0.0), axis=1, keepdims=True)
    w2 = jnp.sum(jnp.where(iota == e2, scores, 0.0), axis=1, keepdims=True)
    denom = w1 + w2 + 1e-20
    w1n = w1 / denom * RSF
    w2n = w2 / denom * RSF
    return (jnp.where(iota == e1, w1n, 0.0)
            + jnp.where(iota == e2, w2n, 0.0))


def _moe_kernel(x_ref, gw_ref, bias_ref, wg_ref, wu_ref, wd_ref,
                swg_ref, swu_ref, swd_ref, out_ref, we_ref):
    e = pl.program_id(0)
    ki = pl.program_id(1)
    x = x_ref[...]

    @pl.when((e == 0) & (ki == 0))
    def _first():
        we_ref[...] = _routing_combine(x, gw_ref[...], bias_ref[...])
        sg = _bf16_dot(x, swg_ref[...], _DN_T)
        su = _bf16_dot(x, swu_ref[...], _DN_T)
        out_ref[...] = _bf16_dot(_silu(sg) * su, swd_ref[...], _DN_T)

    g = _bf16_dot(x, wg_ref[0], _DN_T)
    u = _bf16_dot(x, wu_ref[0], _DN_T)
    h = _silu(g) * u
    y = _bf16_dot(h, wd_ref[0], _DN_T)
    iota = jax.lax.broadcasted_iota(jnp.int32, we_ref.shape, 1)
    w_col = jnp.sum(jnp.where(iota == e, we_ref[...], 0.0), axis=1,
                    keepdims=True)
    out_ref[...] += w_col * y


def kernel(hidden_states, gate_weight, correction_bias, w_gate, w_up, w_down,
           sw_gate, sw_up, sw_down, num_global_tokens, max_num_tokens_per_gpu):
    T = hidden_states.shape[0]
    bias2d = correction_bias.reshape(1, E)
    n_ki = I // IB
    grid = (E, n_ki)
    return pl.pallas_call(
        _moe_kernel,
        grid=grid,
        in_specs=[
            pl.BlockSpec((T, H), lambda e, k: (0, 0)),
            pl.BlockSpec((E, H), lambda e, k: (0, 0)),
            pl.BlockSpec((1, E), lambda e, k: (0, 0)),
            pl.BlockSpec((1, IB, H), lambda e, k: (e, k, 0)),
            pl.BlockSpec((1, IB, H), lambda e, k: (e, k, 0)),
            pl.BlockSpec((1, H, IB), lambda e, k: (e, 0, k)),
            pl.BlockSpec((I, H), lambda e, k: (0, 0)),
            pl.BlockSpec((I, H), lambda e, k: (0, 0)),
            pl.BlockSpec((H, I), lambda e, k: (0, 0)),
        ],
        out_specs=pl.BlockSpec((T, H), lambda e, k: (0, 0)),
        out_shape=jax.ShapeDtypeStruct((T, H), jnp.float32),
        scratch_shapes=[pltpu.VMEM((T, E), jnp.float32)],
        compiler_params=pltpu.CompilerParams(
            dimension_semantics=("arbitrary", "arbitrary")),
    )(hidden_states, gate_weight, bias2d, w_gate, w_up, w_down,
      sw_gate, sw_up, sw_down)
